# baseline (speedup 1.0000x reference)
import functools

import jax
import jax.numpy as jnp
from jax import lax
from jax.experimental import pallas as pl
from jax.experimental.pallas import tpu as pltpu
from jax.experimental.pallas import tpu_sc as plsc

T = 2048
D_MODEL = 1024
D_FF = 2048
E = 8
EP = 128
NP = 2 * T
TILE = 128
NTILES = 40
NPAD = NTILES * TILE
G = 16
GT = T // G
WSW = 128

NW = 32
PPW = NP // NW
DCH = 4
DCW = PPW // DCH
TPW = T // NW
CCH = 4
CCW = TPW // CCH


def _route_body(x_ref, wg_ref, bg_ref, pos_ref, w_ref, eid_ref):
    f32 = jnp.float32
    logits = jnp.dot(x_ref[...], wg_ref[...],
                     preferred_element_type=f32) + bg_ref[...]
    col = lax.broadcasted_iota(jnp.int32, (T, EP), 1)
    m1 = jnp.max(logits, axis=1, keepdims=True)
    i1 = jnp.min(jnp.where(logits == m1, col, EP), axis=1, keepdims=True)
    masked = jnp.where(col == i1, -jnp.inf, logits)
    m2 = jnp.max(masked, axis=1, keepdims=True)
    i2 = jnp.min(jnp.where(masked == m2, col, EP), axis=1, keepdims=True)
    eb = jnp.exp(m2 - m1)
    z = 1.0 + eb
    w_ref[...] = jnp.concatenate([1.0 / z, eb / z], axis=1)

    onec = ((col == i1) | (col == i2)).astype(f32)
    cnt = jnp.sum(onec, axis=0, keepdims=True)
    ptiles = jnp.floor((cnt + (TILE - 1)) / TILE)
    r = lax.broadcasted_iota(jnp.int32, (EP, EP), 0)
    q = lax.broadcasted_iota(jnp.int32, (EP, EP), 1)
    ustrict = (r < q).astype(f32)
    tstart = jnp.dot(ptiles, ustrict, preferred_element_type=f32)
    tend = tstart + ptiles
    poff = tstart * TILE

    lane = lax.broadcasted_iota(jnp.int32, (EP, EP), 1)
    jrow = lax.broadcasted_iota(jnp.int32, (EP, EP), 0).astype(f32)
    tend_b = jnp.broadcast_to(tend, (EP, EP))
    eid = jnp.sum(jnp.where((lane < E) & (tend_b <= jrow), 1.0, 0.0),
                  axis=1, keepdims=True)
    eid_ref[...] = jnp.minimum(eid, E - 1).astype(jnp.int32)

    oneg = onec.reshape(G, GT, EP)
    gs = jnp.sum(oneg, axis=1)
    rg = lax.broadcasted_iota(jnp.int32, (G, G), 0)
    qg = lax.broadcasted_iota(jnp.int32, (G, G), 1)
    l16 = (qg < rg).astype(f32)
    gcs = jnp.dot(l16, gs, preferred_element_type=f32)
    rt = lax.broadcasted_iota(jnp.int32, (GT, GT), 0)
    qt = lax.broadcasted_iota(jnp.int32, (GT, GT), 1)
    l128 = (qt < rt).astype(f32)
    colg = lax.broadcasted_iota(jnp.int32, (GT, EP), 1)
    poff_b = jnp.broadcast_to(poff, (GT, EP))
    for g in range(G):
        sg = (jnp.dot(l128, oneg[g], preferred_element_type=f32)
              + gcs[g:g + 1, :] + poff_b)
        i1g = i1[g * GT:(g + 1) * GT]
        i2g = i2[g * GT:(g + 1) * GT]
        p1 = jnp.sum(jnp.where(colg == i1g, sg, 0.0), axis=1, keepdims=True)
        p2 = jnp.sum(jnp.where(colg == i2g, sg, 0.0), axis=1, keepdims=True)
        pos_ref[g * GT:(g + 1) * GT, :] = jnp.concatenate(
            [p1, p2], axis=1).astype(jnp.int32)


def _gffn_body(eid_ref, xs_ref, w1_ref, b1_ref, w2_ref, b2_ref, ws2_ref,
               out_ref):
    del eid_ref
    h = jnp.maximum(
        jnp.dot(xs_ref[...].astype(jnp.bfloat16),
                w1_ref[0].astype(jnp.bfloat16),
                preferred_element_type=jnp.float32) + b1_ref[0], 0.0)
    y = (jnp.dot(h.astype(jnp.bfloat16), w2_ref[0].astype(jnp.bfloat16),
                 preferred_element_type=jnp.float32) + b2_ref[0])
    out_ref[...] = ws2_ref[...][:, 0:1] * y


def _dispatch_body(x_hbm, pos3_hbm, tid3_hbm, wsrc_hbm, xs_hbm, ws2_hbm,
                   pos_v, tid_v, rows0, rows1, wv0, wv1,
                   sg0, sg1, ss0, ss1, sw0, sw1):
    wid = lax.axis_index("s") * 2 + lax.axis_index("c")
    pltpu.sync_copy(pos3_hbm.at[wid], pos_v)
    pltpu.sync_copy(tid3_hbm.at[wid], tid_v)
    base = wid * PPW
    rows = (rows0, rows1)
    wv = (wv0, wv1)
    sg = (sg0, sg1)
    ss = (ss0, ss1)
    sw = (sw0, sw1)
    scat = [None, None]
    wscat = [None, None]
    gath = [None, None]
    gath[0] = pltpu.async_copy(x_hbm.at[tid_v.at[0]], rows[0], sg[0])
    pltpu.sync_copy(wsrc_hbm.at[pl.ds(base, DCW)], wv[0])
    for c in range(DCH):
        b = c % 2
        if c + 1 < DCH:
            b2 = (c + 1) % 2
            if scat[b2] is not None:
                scat[b2].wait()
                wscat[b2].wait()
            gath[b2] = pltpu.async_copy(
                x_hbm.at[tid_v.at[c + 1]], rows[b2], sg[b2])
            pltpu.sync_copy(
                wsrc_hbm.at[pl.ds(base + (c + 1) * DCW, DCW)], wv[b2])
        gath[b].wait()
        scat[b] = pltpu.async_copy(rows[b], xs_hbm.at[pos_v.at[c]], ss[b])
        wscat[b] = pltpu.async_copy(wv[b], ws2_hbm.at[pos_v.at[c]], sw[b])
    for b in range(2):
        if scat[b] is not None:
            scat[b].wait()
            wscat[b].wait()


def _combine_body(ysw_hbm, pe3_hbm, po3_hbm, out_hbm,
                  pe_v, po_v, a0, a1, b0, b1, o0, o1,
                  sa0, sa1, sb0, sb1, so0, so1):
    wid = lax.axis_index("s") * 2 + lax.axis_index("c")
    pltpu.sync_copy(pe3_hbm.at[wid], pe_v)
    pltpu.sync_copy(po3_hbm.at[wid], po_v)
    tbase = wid * TPW
    av = (a0, a1)
    bv = (b0, b1)
    ov = (o0, o1)
    sa = (sa0, sa1)
    sb = (sb0, sb1)
    so = (so0, so1)
    ga = [None, None]
    gb = [None, None]
    ost = [None, None]
    ga[0] = pltpu.async_copy(ysw_hbm.at[pe_v.at[0]], av[0], sa[0])
    gb[0] = pltpu.async_copy(ysw_hbm.at[po_v.at[0]], bv[0], sb[0])
    for c in range(CCH):
        b = c % 2
        if c + 1 < CCH:
            b2 = (c + 1) % 2
            if ost[b2] is not None:
                ost[b2].wait()
            ga[b2] = pltpu.async_copy(ysw_hbm.at[pe_v.at[c + 1]], av[b2],
                                      sa[b2])
            gb[b2] = pltpu.async_copy(ysw_hbm.at[po_v.at[c + 1]], bv[b2],
                                      sb[b2])
        ga[b].wait()
        gb[b].wait()
        abuf, bbuf, obuf = av[b], bv[b], ov[b]

        def _row(i, _):
            def _lane(j, _):
                obuf[i, pl.ds(j * 16, 16)] = (
                    abuf[i, pl.ds(j * 16, 16)] + bbuf[i, pl.ds(j * 16, 16)])
                return 0
            lax.fori_loop(0, D_MODEL // 16, _lane, 0)
            return 0
        lax.fori_loop(0, CCW, _row, 0)
        ost[b] = pltpu.async_copy(
            obuf, out_hbm.at[pl.ds(tbase + c * CCW, CCW)], so[b])
    for b in range(2):
        if ost[b] is not None:
            ost[b].wait()


def kernel(x, Wg, bg, W1, b1, W2, b2):
    f32 = jnp.float32
    wg_p = jnp.pad(Wg, ((0, 0), (0, EP - E)))
    bg_p = jnp.full((1, EP), -1e30, f32).at[0, :E].set(bg)

    pos, w, eid = pl.pallas_call(
        _route_body,
        out_shape=(
            jax.ShapeDtypeStruct((T, 2), jnp.int32),
            jax.ShapeDtypeStruct((T, 2), f32),
            jax.ShapeDtypeStruct((EP, 1), jnp.int32),
        ),
    )(x, wg_p, bg_p)

    pos_flat = pos.reshape(NP)
    pos3 = pos_flat.reshape(NW, DCH, DCW)
    tid3 = (jnp.arange(NP, dtype=jnp.int32) // 2).reshape(NW, DCH, DCW)
    wsrc = jnp.broadcast_to(w.reshape(NP, 1), (NP, WSW))
    eid1d = eid.reshape(EP)[:NTILES]

    mesh = plsc.VectorSubcoreMesh(core_axis_name="c", subcore_axis_name="s")
    xs, ws2 = pl.kernel(
        _dispatch_body,
        out_type=(
            jax.ShapeDtypeStruct((NPAD, D_MODEL), f32),
            jax.ShapeDtypeStruct((NPAD, WSW), f32),
        ),
        mesh=mesh,
        scratch_types=[
            pltpu.VMEM((DCH, DCW), jnp.int32),
            pltpu.VMEM((DCH, DCW), jnp.int32),
            pltpu.VMEM((DCW, D_MODEL), f32),
            pltpu.VMEM((DCW, D_MODEL), f32),
            pltpu.VMEM((DCW, WSW), f32),
            pltpu.VMEM((DCW, WSW), f32),
        ] + [pltpu.SemaphoreType.DMA] * 6,
    )(x, pos3, tid3, wsrc)

    grid_spec = pltpu.PrefetchScalarGridSpec(
        num_scalar_prefetch=1,
        grid=(NTILES,),
        in_specs=[
            pl.BlockSpec((TILE, D_MODEL), lambda j, eid: (j, 0)),
            pl.BlockSpec((1, D_MODEL, D_FF), lambda j, eid: (eid[j], 0, 0)),
            pl.BlockSpec((1, 1, D_FF), lambda j, eid: (eid[j], 0, 0)),
            pl.BlockSpec((1, D_FF, D_MODEL), lambda j, eid: (eid[j], 0, 0)),
            pl.BlockSpec((1, 1, D_MODEL), lambda j, eid: (eid[j], 0, 0)),
            pl.BlockSpec((TILE, WSW), lambda j, eid: (j, 0)),
        ],
        out_specs=pl.BlockSpec((TILE, D_MODEL), lambda j, eid: (j, 0)),
    )
    ysw = pl.pallas_call(
        _gffn_body,
        grid_spec=grid_spec,
        out_shape=jax.ShapeDtypeStruct((NPAD, D_MODEL), f32),
    )(eid1d, xs, W1, b1.reshape(E, 1, D_FF), W2,
      b2.reshape(E, 1, D_MODEL), ws2)

    pe3 = pos[:, 0].reshape(NW, CCH, CCW)
    po3 = pos[:, 1].reshape(NW, CCH, CCW)
    out = pl.kernel(
        _combine_body,
        out_type=jax.ShapeDtypeStruct((T, D_MODEL), f32),
        mesh=mesh,
        scratch_types=[
            pltpu.VMEM((CCH, CCW), jnp.int32),
            pltpu.VMEM((CCH, CCW), jnp.int32),
        ] + [pltpu.VMEM((CCW, D_MODEL), f32)] * 6
          + [pltpu.SemaphoreType.DMA] * 6,
    )(ysw, pe3, po3)
    return out


# speedup vs baseline: 1.0207x; 1.0207x over previous
import functools

import jax
import jax.numpy as jnp
from jax import lax
from jax.experimental import pallas as pl
from jax.experimental.pallas import tpu as pltpu
from jax.experimental.pallas import tpu_sc as plsc

T = 2048
D_MODEL = 1024
D_FF = 2048
E = 8
EP = 128
NP = 2 * T
TILE = 256
NTILES = 24
NPAD = NTILES * TILE
G = 16
GT = T // G
WSW = 128

NW = 32
PPW = NP // NW
DCH = 4
DCW = PPW // DCH
TPW = T // NW
CCH = 4
CCW = TPW // CCH


def _route_body(x_ref, wg_ref, bg_ref, pos_ref, w_ref, eid_ref):
    f32 = jnp.float32
    logits = jnp.dot(x_ref[...], wg_ref[...],
                     preferred_element_type=f32) + bg_ref[...]
    col = lax.broadcasted_iota(jnp.int32, (T, EP), 1)
    m1 = jnp.max(logits, axis=1, keepdims=True)
    i1 = jnp.min(jnp.where(logits == m1, col, EP), axis=1, keepdims=True)
    masked = jnp.where(col == i1, -jnp.inf, logits)
    m2 = jnp.max(masked, axis=1, keepdims=True)
    i2 = jnp.min(jnp.where(masked == m2, col, EP), axis=1, keepdims=True)
    eb = jnp.exp(m2 - m1)
    z = 1.0 + eb
    w_ref[...] = jnp.concatenate([1.0 / z, eb / z], axis=1)

    onec = ((col == i1) | (col == i2)).astype(f32)
    cnt = jnp.sum(onec, axis=0, keepdims=True)
    ptiles = jnp.floor((cnt + (TILE - 1)) / TILE)
    r = lax.broadcasted_iota(jnp.int32, (EP, EP), 0)
    q = lax.broadcasted_iota(jnp.int32, (EP, EP), 1)
    ustrict = (r < q).astype(f32)
    tstart = jnp.dot(ptiles, ustrict, preferred_element_type=f32)
    tend = tstart + ptiles
    poff = tstart * TILE

    lane = lax.broadcasted_iota(jnp.int32, (EP, EP), 1)
    jrow = lax.broadcasted_iota(jnp.int32, (EP, EP), 0).astype(f32)
    tend_b = jnp.broadcast_to(tend, (EP, EP))
    eid = jnp.sum(jnp.where((lane < E) & (tend_b <= jrow), 1.0, 0.0),
                  axis=1, keepdims=True)
    eid_ref[...] = jnp.minimum(eid, E - 1).astype(jnp.int32)

    oneg = onec.reshape(G, GT, EP)
    gs = jnp.sum(oneg, axis=1)
    rg = lax.broadcasted_iota(jnp.int32, (G, G), 0)
    qg = lax.broadcasted_iota(jnp.int32, (G, G), 1)
    l16 = (qg < rg).astype(f32)
    gcs = jnp.dot(l16, gs, preferred_element_type=f32)
    rt = lax.broadcasted_iota(jnp.int32, (GT, GT), 0)
    qt = lax.broadcasted_iota(jnp.int32, (GT, GT), 1)
    l128 = (qt < rt).astype(f32)
    colg = lax.broadcasted_iota(jnp.int32, (GT, EP), 1)
    poff_b = jnp.broadcast_to(poff, (GT, EP))
    for g in range(G):
        sg = (jnp.dot(l128, oneg[g], preferred_element_type=f32)
              + gcs[g:g + 1, :] + poff_b)
        i1g = i1[g * GT:(g + 1) * GT]
        i2g = i2[g * GT:(g + 1) * GT]
        p1 = jnp.sum(jnp.where(colg == i1g, sg, 0.0), axis=1, keepdims=True)
        p2 = jnp.sum(jnp.where(colg == i2g, sg, 0.0), axis=1, keepdims=True)
        pos_ref[g * GT:(g + 1) * GT, :] = jnp.concatenate(
            [p1, p2], axis=1).astype(jnp.int32)


def _gffn_body(eid_ref, xs_ref, w1_ref, b1_ref, w2_ref, b2_ref, ws2_ref,
               out_ref):
    del eid_ref
    h = jnp.maximum(
        jnp.dot(xs_ref[...].astype(jnp.bfloat16),
                w1_ref[0].astype(jnp.bfloat16),
                preferred_element_type=jnp.float32) + b1_ref[0], 0.0)
    y = (jnp.dot(h.astype(jnp.bfloat16), w2_ref[0].astype(jnp.bfloat16),
                 preferred_element_type=jnp.float32) + b2_ref[0])
    out_ref[...] = ws2_ref[...][:, 0:1] * y


def _dispatch_body(x_hbm, pos3_hbm, tid3_hbm, wsrc_hbm, xs_hbm, ws2_hbm,
                   pos_v, tid_v, rows0, rows1, wv0, wv1,
                   sg0, sg1, ss0, ss1, sw0, sw1):
    wid = lax.axis_index("s") * 2 + lax.axis_index("c")
    pltpu.sync_copy(pos3_hbm.at[wid], pos_v)
    pltpu.sync_copy(tid3_hbm.at[wid], tid_v)
    base = wid * PPW
    rows = (rows0, rows1)
    wv = (wv0, wv1)
    sg = (sg0, sg1)
    ss = (ss0, ss1)
    sw = (sw0, sw1)
    scat = [None, None]
    wscat = [None, None]
    gath = [None, None]
    gath[0] = pltpu.async_copy(x_hbm.at[tid_v.at[0]], rows[0], sg[0])
    pltpu.sync_copy(wsrc_hbm.at[pl.ds(base, DCW)], wv[0])
    for c in range(DCH):
        b = c % 2
        if c + 1 < DCH:
            b2 = (c + 1) % 2
            if scat[b2] is not None:
                scat[b2].wait()
                wscat[b2].wait()
            gath[b2] = pltpu.async_copy(
                x_hbm.at[tid_v.at[c + 1]], rows[b2], sg[b2])
            pltpu.sync_copy(
                wsrc_hbm.at[pl.ds(base + (c + 1) * DCW, DCW)], wv[b2])
        gath[b].wait()
        scat[b] = pltpu.async_copy(rows[b], xs_hbm.at[pos_v.at[c]], ss[b])
        wscat[b] = pltpu.async_copy(wv[b], ws2_hbm.at[pos_v.at[c]], sw[b])
    for b in range(2):
        if scat[b] is not None:
            scat[b].wait()
            wscat[b].wait()


def _combine_body(ysw_hbm, pe3_hbm, po3_hbm, out_hbm,
                  pe_v, po_v, a0, a1, b0, b1, o0, o1,
                  sa0, sa1, sb0, sb1, so0, so1):
    wid = lax.axis_index("s") * 2 + lax.axis_index("c")
    pltpu.sync_copy(pe3_hbm.at[wid], pe_v)
    pltpu.sync_copy(po3_hbm.at[wid], po_v)
    tbase = wid * TPW
    av = (a0, a1)
    bv = (b0, b1)
    ov = (o0, o1)
    sa = (sa0, sa1)
    sb = (sb0, sb1)
    so = (so0, so1)
    ga = [None, None]
    gb = [None, None]
    ost = [None, None]
    ga[0] = pltpu.async_copy(ysw_hbm.at[pe_v.at[0]], av[0], sa[0])
    gb[0] = pltpu.async_copy(ysw_hbm.at[po_v.at[0]], bv[0], sb[0])
    for c in range(CCH):
        b = c % 2
        if c + 1 < CCH:
            b2 = (c + 1) % 2
            if ost[b2] is not None:
                ost[b2].wait()
            ga[b2] = pltpu.async_copy(ysw_hbm.at[pe_v.at[c + 1]], av[b2],
                                      sa[b2])
            gb[b2] = pltpu.async_copy(ysw_hbm.at[po_v.at[c + 1]], bv[b2],
                                      sb[b2])
        ga[b].wait()
        gb[b].wait()
        abuf, bbuf, obuf = av[b], bv[b], ov[b]

        def _row(i, _):
            def _lane(j, _):
                obuf[i, pl.ds(j * 16, 16)] = (
                    abuf[i, pl.ds(j * 16, 16)] + bbuf[i, pl.ds(j * 16, 16)])
                return 0
            lax.fori_loop(0, D_MODEL // 16, _lane, 0)
            return 0
        lax.fori_loop(0, CCW, _row, 0)
        ost[b] = pltpu.async_copy(
            obuf, out_hbm.at[pl.ds(tbase + c * CCW, CCW)], so[b])
    for b in range(2):
        if ost[b] is not None:
            ost[b].wait()


def kernel(x, Wg, bg, W1, b1, W2, b2):
    f32 = jnp.float32
    wg_p = jnp.pad(Wg, ((0, 0), (0, EP - E)))
    bg_p = jnp.full((1, EP), -1e30, f32).at[0, :E].set(bg)

    pos, w, eid = pl.pallas_call(
        _route_body,
        out_shape=(
            jax.ShapeDtypeStruct((T, 2), jnp.int32),
            jax.ShapeDtypeStruct((T, 2), f32),
            jax.ShapeDtypeStruct((EP, 1), jnp.int32),
        ),
    )(x, wg_p, bg_p)

    pos_flat = pos.reshape(NP)
    pos3 = pos_flat.reshape(NW, DCH, DCW)
    tid3 = (jnp.arange(NP, dtype=jnp.int32) // 2).reshape(NW, DCH, DCW)
    wsrc = jnp.broadcast_to(w.reshape(NP, 1), (NP, WSW))
    eid1d = eid.reshape(EP)[:NTILES]

    mesh = plsc.VectorSubcoreMesh(core_axis_name="c", subcore_axis_name="s")
    xs, ws2 = pl.kernel(
        _dispatch_body,
        out_type=(
            jax.ShapeDtypeStruct((NPAD, D_MODEL), f32),
            jax.ShapeDtypeStruct((NPAD, WSW), f32),
        ),
        mesh=mesh,
        scratch_types=[
            pltpu.VMEM((DCH, DCW), jnp.int32),
            pltpu.VMEM((DCH, DCW), jnp.int32),
            pltpu.VMEM((DCW, D_MODEL), f32),
            pltpu.VMEM((DCW, D_MODEL), f32),
            pltpu.VMEM((DCW, WSW), f32),
            pltpu.VMEM((DCW, WSW), f32),
        ] + [pltpu.SemaphoreType.DMA] * 6,
    )(x, pos3, tid3, wsrc)

    grid_spec = pltpu.PrefetchScalarGridSpec(
        num_scalar_prefetch=1,
        grid=(NTILES,),
        in_specs=[
            pl.BlockSpec((TILE, D_MODEL), lambda j, eid: (j, 0)),
            pl.BlockSpec((1, D_MODEL, D_FF), lambda j, eid: (eid[j], 0, 0)),
            pl.BlockSpec((1, 1, D_FF), lambda j, eid: (eid[j], 0, 0)),
            pl.BlockSpec((1, D_FF, D_MODEL), lambda j, eid: (eid[j], 0, 0)),
            pl.BlockSpec((1, 1, D_MODEL), lambda j, eid: (eid[j], 0, 0)),
            pl.BlockSpec((TILE, WSW), lambda j, eid: (j, 0)),
        ],
        out_specs=pl.BlockSpec((TILE, D_MODEL), lambda j, eid: (j, 0)),
    )
    ysw = pl.pallas_call(
        _gffn_body,
        grid_spec=grid_spec,
        out_shape=jax.ShapeDtypeStruct((NPAD, D_MODEL), f32),
    )(eid1d, xs, W1, b1.reshape(E, 1, D_FF), W2,
      b2.reshape(E, 1, D_MODEL), ws2)

    pe3 = pos[:, 0].reshape(NW, CCH, CCW)
    po3 = pos[:, 1].reshape(NW, CCH, CCW)
    out = pl.kernel(
        _combine_body,
        out_type=jax.ShapeDtypeStruct((T, D_MODEL), f32),
        mesh=mesh,
        scratch_types=[
            pltpu.VMEM((CCH, CCW), jnp.int32),
            pltpu.VMEM((CCH, CCW), jnp.int32),
        ] + [pltpu.VMEM((CCW, D_MODEL), f32)] * 6
          + [pltpu.SemaphoreType.DMA] * 6,
    )(ysw, pe3, po3)
    return out


# speedup vs baseline: 1.0208x; 1.0001x over previous
import jax
import jax.numpy as jnp
from jax import lax
from jax.experimental import pallas as pl
from jax.experimental.pallas import tpu as pltpu
from jax.experimental.pallas import tpu_sc as plsc

T = 2048
D_MODEL = 1024
D_FF = 2048
E = 8
EP = 128
NP = 2 * T
TILE = 256
NTILES = 24
NPAD = NTILES * TILE
G = 16
GT = T // G
WSW = 128

NW = 32
PPW = NP // NW
DCH = 4
DCW = PPW // DCH
TPW = T // NW
CCH = 4
CCW = TPW // CCH


def _route_body(x_ref, wg_ref, bg_ref, pos_ref, w_ref, eid_ref):
    f32 = jnp.float32
    logits = jnp.dot(x_ref[...], wg_ref[...],
                     preferred_element_type=f32) + bg_ref[...]
    col = lax.broadcasted_iota(jnp.int32, (T, EP), 1)
    m1 = jnp.max(logits, axis=1, keepdims=True)
    i1 = jnp.min(jnp.where(logits == m1, col, EP), axis=1, keepdims=True)
    masked = jnp.where(col == i1, -jnp.inf, logits)
    m2 = jnp.max(masked, axis=1, keepdims=True)
    i2 = jnp.min(jnp.where(masked == m2, col, EP), axis=1, keepdims=True)
    eb = jnp.exp(m2 - m1)
    z = 1.0 + eb
    w_ref[...] = jnp.concatenate([1.0 / z, eb / z], axis=1)

    onec = ((col == i1) | (col == i2)).astype(f32)
    cnt = jnp.sum(onec, axis=0, keepdims=True)
    ptiles = jnp.floor((cnt + (TILE - 1)) / TILE)
    r = lax.broadcasted_iota(jnp.int32, (EP, EP), 0)
    q = lax.broadcasted_iota(jnp.int32, (EP, EP), 1)
    ustrict = (r < q).astype(f32)
    tstart = jnp.dot(ptiles, ustrict, preferred_element_type=f32)
    tend = tstart + ptiles
    poff = tstart * TILE

    lane = lax.broadcasted_iota(jnp.int32, (EP, EP), 1)
    jrow = lax.broadcasted_iota(jnp.int32, (EP, EP), 0).astype(f32)
    tend_b = jnp.broadcast_to(tend, (EP, EP))
    eid = jnp.sum(jnp.where((lane < E) & (tend_b <= jrow), 1.0, 0.0),
                  axis=1, keepdims=True)
    eid_ref[...] = jnp.minimum(eid, E - 1).astype(jnp.int32)

    oneg = onec.reshape(G, GT, EP)
    gs = jnp.sum(oneg, axis=1)
    rg = lax.broadcasted_iota(jnp.int32, (G, G), 0)
    qg = lax.broadcasted_iota(jnp.int32, (G, G), 1)
    l16 = (qg < rg).astype(f32)
    gcs = jnp.dot(l16, gs, preferred_element_type=f32)
    rt = lax.broadcasted_iota(jnp.int32, (GT, GT), 0)
    qt = lax.broadcasted_iota(jnp.int32, (GT, GT), 1)
    l128 = (qt < rt).astype(f32)
    colg = lax.broadcasted_iota(jnp.int32, (GT, EP), 1)
    poff_b = jnp.broadcast_to(poff, (GT, EP))
    for g in range(G):
        sg = (jnp.dot(l128, oneg[g], preferred_element_type=f32)
              + gcs[g:g + 1, :] + poff_b)
        i1g = i1[g * GT:(g + 1) * GT]
        i2g = i2[g * GT:(g + 1) * GT]
        p1 = jnp.sum(jnp.where(colg == i1g, sg, 0.0), axis=1, keepdims=True)
        p2 = jnp.sum(jnp.where(colg == i2g, sg, 0.0), axis=1, keepdims=True)
        pos_ref[g * GT:(g + 1) * GT, :] = jnp.concatenate(
            [p1, p2], axis=1).astype(jnp.int32)


def _gffn_body(eid_ref, xs_ref, w1_ref, b1_ref, w2_ref, b2_ref, ws2_ref,
               out_ref):
    del eid_ref
    h = jnp.maximum(
        jnp.dot(xs_ref[...].astype(jnp.bfloat16),
                w1_ref[0].astype(jnp.bfloat16),
                preferred_element_type=jnp.float32) + b1_ref[0], 0.0)
    y = (jnp.dot(h.astype(jnp.bfloat16), w2_ref[0].astype(jnp.bfloat16),
                 preferred_element_type=jnp.float32) + b2_ref[0])
    out_ref[...] = ws2_ref[...][:, 0:1] * y


def _dispatch_body(x_hbm, pos3_hbm, tid3_hbm, wsrc_hbm, xs_hbm, ws2_hbm,
                   pos_v, tid_v, rows0, rows1, wv0, wv1,
                   sg0, sg1, ss0, ss1, sw0, sw1):
    wid = lax.axis_index("s") * 2 + lax.axis_index("c")
    pltpu.sync_copy(pos3_hbm.at[wid], pos_v)
    pltpu.sync_copy(tid3_hbm.at[wid], tid_v)
    base = wid * PPW
    rows = (rows0, rows1)
    wv = (wv0, wv1)
    sg = (sg0, sg1)
    ss = (ss0, ss1)
    sw = (sw0, sw1)
    scat = [None, None]
    wscat = [None, None]
    gath = [None, None]
    gath[0] = pltpu.async_copy(x_hbm.at[tid_v.at[0]], rows[0], sg[0])
    pltpu.sync_copy(wsrc_hbm.at[pl.ds(base, DCW)], wv[0])
    for c in range(DCH):
        b = c % 2
        if c + 1 < DCH:
            b2 = (c + 1) % 2
            if scat[b2] is not None:
                scat[b2].wait()
                wscat[b2].wait()
            gath[b2] = pltpu.async_copy(
                x_hbm.at[tid_v.at[c + 1]], rows[b2], sg[b2])
            pltpu.sync_copy(
                wsrc_hbm.at[pl.ds(base + (c + 1) * DCW, DCW)], wv[b2])
        gath[b].wait()
        scat[b] = pltpu.async_copy(rows[b], xs_hbm.at[pos_v.at[c]], ss[b])
        wscat[b] = pltpu.async_copy(wv[b], ws2_hbm.at[pos_v.at[c]], sw[b])
    for b in range(2):
        if scat[b] is not None:
            scat[b].wait()
            wscat[b].wait()


def _combine_body(ysw_hbm, pe3_hbm, po3_hbm, out_hbm,
                  pe_v, po_v, a0, a1, b0, b1, o0, o1,
                  sa0, sa1, sb0, sb1, so0, so1):
    wid = lax.axis_index("s") * 2 + lax.axis_index("c")
    pltpu.sync_copy(pe3_hbm.at[wid], pe_v)
    pltpu.sync_copy(po3_hbm.at[wid], po_v)
    tbase = wid * TPW
    av = (a0, a1)
    bv = (b0, b1)
    ov = (o0, o1)
    sa = (sa0, sa1)
    sb = (sb0, sb1)
    so = (so0, so1)
    ga = [None, None]
    gb = [None, None]
    ost = [None, None]
    ga[0] = pltpu.async_copy(ysw_hbm.at[pe_v.at[0]], av[0], sa[0])
    gb[0] = pltpu.async_copy(ysw_hbm.at[po_v.at[0]], bv[0], sb[0])
    for c in range(CCH):
        b = c % 2
        if c + 1 < CCH:
            b2 = (c + 1) % 2
            if ost[b2] is not None:
                ost[b2].wait()
            ga[b2] = pltpu.async_copy(ysw_hbm.at[pe_v.at[c + 1]], av[b2],
                                      sa[b2])
            gb[b2] = pltpu.async_copy(ysw_hbm.at[po_v.at[c + 1]], bv[b2],
                                      sb[b2])
        ga[b].wait()
        gb[b].wait()
        abuf, bbuf, obuf = av[b], bv[b], ov[b]

        def _row(i, _):
            def _lane(j, _):
                obuf[i, pl.ds(j * 16, 16)] = (
                    abuf[i, pl.ds(j * 16, 16)] + bbuf[i, pl.ds(j * 16, 16)])
                return 0
            lax.fori_loop(0, D_MODEL // 16, _lane, 0)
            return 0
        lax.fori_loop(0, CCW, _row, 0)
        ost[b] = pltpu.async_copy(
            obuf, out_hbm.at[pl.ds(tbase + c * CCW, CCW)], so[b])
    for b in range(2):
        if ost[b] is not None:
            ost[b].wait()


def kernel(x, Wg, bg, W1, b1, W2, b2):
    f32 = jnp.float32
    wg_p = jnp.pad(Wg, ((0, 0), (0, EP - E)))
    bg_p = jnp.full((1, EP), -1e30, f32).at[0, :E].set(bg)

    pos, w, eid = pl.pallas_call(
        _route_body,
        out_shape=(
            jax.ShapeDtypeStruct((T, 2), jnp.int32),
            jax.ShapeDtypeStruct((T, 2), f32),
            jax.ShapeDtypeStruct((EP, 1), jnp.int32),
        ),
    )(x, wg_p, bg_p)

    pos_flat = pos.reshape(NP)
    pos3 = pos_flat.reshape(NW, DCH, DCW)
    tid3 = (jnp.arange(NP, dtype=jnp.int32) // 2).reshape(NW, DCH, DCW)
    wsrc = jnp.broadcast_to(w.reshape(NP, 1), (NP, WSW))
    eid1d = eid.reshape(EP)[:NTILES]

    mesh = plsc.VectorSubcoreMesh(core_axis_name="c", subcore_axis_name="s")
    xs, ws2 = pl.kernel(
        _dispatch_body,
        out_type=(
            jax.ShapeDtypeStruct((NPAD, D_MODEL), f32),
            jax.ShapeDtypeStruct((NPAD, WSW), f32),
        ),
        mesh=mesh,
        scratch_types=[
            pltpu.VMEM((DCH, DCW), jnp.int32),
            pltpu.VMEM((DCH, DCW), jnp.int32),
            pltpu.VMEM((DCW, D_MODEL), f32),
            pltpu.VMEM((DCW, D_MODEL), f32),
            pltpu.VMEM((DCW, WSW), f32),
            pltpu.VMEM((DCW, WSW), f32),
        ] + [pltpu.SemaphoreType.DMA] * 6,
    )(x, pos3, tid3, wsrc)

    grid_spec = pltpu.PrefetchScalarGridSpec(
        num_scalar_prefetch=1,
        grid=(NTILES,),
        in_specs=[
            pl.BlockSpec((TILE, D_MODEL), lambda j, eid: (j, 0)),
            pl.BlockSpec((1, D_MODEL, D_FF), lambda j, eid: (eid[j], 0, 0)),
            pl.BlockSpec((1, 1, D_FF), lambda j, eid: (eid[j], 0, 0)),
            pl.BlockSpec((1, D_FF, D_MODEL), lambda j, eid: (eid[j], 0, 0)),
            pl.BlockSpec((1, 1, D_MODEL), lambda j, eid: (eid[j], 0, 0)),
            pl.BlockSpec((TILE, WSW), lambda j, eid: (j, 0)),
        ],
        out_specs=pl.BlockSpec((TILE, D_MODEL), lambda j, eid: (j, 0)),
    )
    ysw = pl.pallas_call(
        _gffn_body,
        grid_spec=grid_spec,
        out_shape=jax.ShapeDtypeStruct((NPAD, D_MODEL), f32),
    )(eid1d, xs, W1, b1.reshape(E, 1, D_FF), W2,
      b2.reshape(E, 1, D_MODEL), ws2)

    pe3 = pos[:, 0].reshape(NW, CCH, CCW)
    po3 = pos[:, 1].reshape(NW, CCH, CCW)
    out = pl.kernel(
        _combine_body,
        out_type=jax.ShapeDtypeStruct((T, D_MODEL), f32),
        mesh=mesh,
        scratch_types=[
            pltpu.VMEM((CCH, CCW), jnp.int32),
            pltpu.VMEM((CCH, CCW), jnp.int32),
        ] + [pltpu.VMEM((CCW, D_MODEL), f32)] * 6
          + [pltpu.SemaphoreType.DMA] * 6,
    )(ysw, pe3, po3)
    return out
